# Initial kernel scaffold; baseline (speedup 1.0000x reference)
import jax
import jax.numpy as jnp
from jax.experimental import pallas as pl


def kernel(pos, x, tpl_edge_index, geo_edge_index, batch, Wt1, bt1, Wg1, bg1, Wm1, bm1, Wt2, bt2, Wg2, bg2, Wm2, bm2, Wt3, bt3, Wg3, bg3, Wm3, bm3, Wglb, bglb, Wtr, btr):
    raise NotImplementedError("write your pallas kernel here")


# speedup vs baseline: 1.6872x; 1.6872x over previous
import functools

import jax
import jax.numpy as jnp
from jax import lax
from jax.experimental import pallas as pl
from jax.experimental.pallas import tpu as pltpu
from jax.experimental.pallas import tpu_sc as plsc

F32 = jnp.float32
I32 = jnp.int32
NEG = -1e30
NP = 10240
BN = 512
NB = NP // BN
ECH = 2000


def _dot(a, b):
    return lax.dot_general(a, b, (((1,), (0,)), ((), ())),
                           preferred_element_type=F32)


def _nspec(c):
    return pl.BlockSpec((c, BN), lambda i: (0, i))


def _wspec(shape):
    return pl.BlockSpec(shape, lambda i: (0, 0))


def _t0_body(xin_ref, wat, wbt, wag, wbg, bt, bg, at_o, bt_o, ag_o, bg_o):
    X = xin_ref[...]
    at_o[...] = _dot(wat[...], X) + bt[...]
    bt_o[...] = _dot(wbt[...], X)
    ag_o[...] = _dot(wag[...], X) + bg[...]
    bg_o[...] = _dot(wbg[...], X)


def _t0(xinT, WAt, WBt, WAg, WBg, bt, bg):
    co = WAt.shape[0]
    return pl.pallas_call(
        _t0_body,
        grid=(NB,),
        in_specs=[_nspec(xinT.shape[0]), _wspec(WAt.shape), _wspec(WBt.shape),
                  _wspec(WAg.shape), _wspec(WBg.shape), _wspec(bt.shape),
                  _wspec(bg.shape)],
        out_specs=[_nspec(co)] * 4,
        out_shape=[jax.ShapeDtypeStruct((co, NP), F32)] * 4,
    )(xinT, WAt, WBt, WAg, WBg, bt, bg)


def _tmid_body(at, ct, ag, cg, wm, bm, wat, bt2, wbt, wag, bg2, wbg,
               x_o, a2t_o, b2t_o, a2g_o, b2g_o):
    h = jnp.concatenate([jnp.maximum(at[...] + ct[...], 0.0),
                         jnp.maximum(ag[...] + cg[...], 0.0)], axis=0)
    xn = jnp.maximum(_dot(wm[...], h) + bm[...], 0.0)
    x_o[...] = xn
    a2t_o[...] = _dot(wat[...], xn) + bt2[...]
    b2t_o[...] = _dot(wbt[...], xn)
    a2g_o[...] = _dot(wag[...], xn) + bg2[...]
    b2g_o[...] = _dot(wbg[...], xn)


def _tmid(at, ct, ag, cg, wm, bm, wat, bt2, wbt, wag, bg2, wbg):
    ci = at.shape[0]
    cx = wm.shape[0]
    co = wat.shape[0]
    return pl.pallas_call(
        _tmid_body,
        grid=(NB,),
        in_specs=[_nspec(ci)] * 4 + [_wspec(wm.shape), _wspec(bm.shape),
                  _wspec(wat.shape), _wspec(bt2.shape), _wspec(wbt.shape),
                  _wspec(wag.shape), _wspec(bg2.shape), _wspec(wbg.shape)],
        out_specs=[_nspec(cx)] + [_nspec(co)] * 4,
        out_shape=[jax.ShapeDtypeStruct((cx, NP), F32)] +
                  [jax.ShapeDtypeStruct((co, NP), F32)] * 4,
    )(at, ct, ag, cg, wm, bm, wat, bt2, wbt, wag, bg2, wbg)


def _t3_body(at, ct, ag, cg, wm3, bm3, x1, x2, wg1, wg2, wg3, bglb, bb_ref,
             x3_o, xg_o):
    i = pl.program_id(0)
    h = jnp.concatenate([jnp.maximum(at[...] + ct[...], 0.0),
                         jnp.maximum(ag[...] + cg[...], 0.0)], axis=0)
    x3 = jnp.maximum(_dot(wm3[...], h) + bm3[...], 0.0)
    x3_o[...] = x3
    x4 = jnp.maximum(_dot(wg1[...], x1[...]) + _dot(wg2[...], x2[...]) +
                     _dot(wg3[...], x3) + bglb[...], 0.0)

    @pl.when(i == 0)
    def _():
        xg_o[...] = jnp.full((1024, 128), NEG, F32)

    bb = bb_ref[...]
    lane = lax.broadcasted_iota(I32, (1024, 128), 1)
    acc = xg_o[...]
    for b in range(4):
        colmax = jnp.max(jnp.where(bb == b, x4, NEG), axis=1, keepdims=True)
        acc = jnp.maximum(acc, jnp.where(lane == b, colmax, NEG))
    xg_o[...] = acc


def _t3(at, ct, ag, cg, wm3, bm3, x1, x2, wg1, wg2, wg3, bglb, batchP):
    return pl.pallas_call(
        _t3_body,
        grid=(NB,),
        in_specs=[_nspec(256)] * 4 + [_wspec(wm3.shape), _wspec(bm3.shape),
                  _nspec(64), _nspec(256), _wspec(wg1.shape),
                  _wspec(wg2.shape), _wspec(wg3.shape), _wspec(bglb.shape),
                  _nspec(1)],
        out_specs=[_nspec(512), pl.BlockSpec((1024, 128), lambda i: (0, 0))],
        out_shape=[jax.ShapeDtypeStruct((512, NP), F32),
                   jax.ShapeDtypeStruct((1024, 128), F32)],
    )(at, ct, ag, cg, wm3, bm3, x1, x2, wg1, wg2, wg3, bglb, batchP)


def _t4_body(xgm_ref, bb_ref, xin, x1, x2, x3, wg, wxin, w1, w2, w3, btr,
             sig_o, den_o):
    i = pl.program_id(0)
    xgm = xgm_ref[...]
    xg = jnp.where(xgm > -1e29, xgm, 0.0)
    bb = bb_ref[...]
    xgb = jnp.zeros((1024, BN), F32)
    for b in range(4):
        xgb = xgb + jnp.where(bb == b, xg[:, b:b + 1], 0.0)
    z = (_dot(wg[...], xgb) + _dot(wxin[...], xin[...]) +
         _dot(w1[...], x1[...]) + _dot(w2[...], x2[...]) +
         _dot(w3[...], x3[...]) + btr[...])
    sig = 1.0 / (1.0 + jnp.exp(-z))
    sig_o[...] = sig

    @pl.when(i == 0)
    def _():
        den_o[...] = jnp.zeros((24, 128), F32)

    lane = lax.broadcasted_iota(I32, (24, 128), 1)
    acc = den_o[...]
    for b in range(4):
        colsum = jnp.sum(jnp.where(bb == b, sig, 0.0), axis=1, keepdims=True)
        acc = acc + jnp.where(lane == b, colsum, 0.0)
    den_o[...] = acc


def _t4(xgmax, batchP, xinT, x1, x2, x3, wg, wxin, w1, w2, w3, btr):
    return pl.pallas_call(
        _t4_body,
        grid=(NB,),
        in_specs=[pl.BlockSpec((1024, 128), lambda i: (0, 0)), _nspec(1),
                  _nspec(8), _nspec(64), _nspec(256), _nspec(512),
                  _wspec(wg.shape), _wspec(wxin.shape), _wspec(w1.shape),
                  _wspec(w2.shape), _wspec(w3.shape), _wspec(btr.shape)],
        out_specs=[_nspec(24), pl.BlockSpec((24, 128), lambda i: (0, 0))],
        out_shape=[jax.ShapeDtypeStruct((24, NP), F32),
                   jax.ShapeDtypeStruct((24, 128), F32)],
    )(xgmax, batchP, xinT, x1, x2, x3, wg, wxin, w1, w2, w3, btr)


def _t5_body(sig_ref, den_ref, bb_ref, pos_ref, w_o, outs_o):
    i = pl.program_id(0)
    den = den_ref[...]
    recip = 1.0 / (den + 1e-9)
    bb = bb_ref[...]
    rsel = jnp.zeros((24, BN), F32)
    for b in range(4):
        rsel = rsel + jnp.where(bb == b, recip[:, b:b + 1], 0.0)
    w = sig_ref[...] * rsel
    w_o[...] = w

    @pl.when(i == 0)
    def _():
        outs_o[...] = jnp.zeros((96, 128), F32)

    acc = outs_o[...]
    pos = pos_ref[...]
    parts = []
    for b in range(4):
        mw = jnp.where(bb == b, w, 0.0)
        parts.append(_dot(mw, pos))
    outs_o[...] = acc + jnp.concatenate(parts, axis=0)


def _t5(sigT, den, batchP, pos128):
    return pl.pallas_call(
        _t5_body,
        grid=(NB,),
        in_specs=[_nspec(24), pl.BlockSpec((24, 128), lambda i: (0, 0)),
                  _nspec(1), pl.BlockSpec((BN, 128), lambda i: (i, 0))],
        out_specs=[_nspec(24), pl.BlockSpec((96, 128), lambda i: (0, 0))],
        out_shape=[jax.ShapeDtypeStruct((24, NP), F32),
                   jax.ShapeDtypeStruct((96, 128), F32)],
    )(sigT, den, batchP, pos128)


def _sc_segmax(ch, K, E):
    slots = ch // K
    passes = -(-slots // 32)
    nvec = ECH // 16
    nchunk = E // ECH
    mesh = plsc.VectorSubcoreMesh(core_axis_name="c", subcore_axis_name="s")

    @functools.partial(
        pl.kernel,
        out_type=jax.ShapeDtypeStruct((ch, NP), F32),
        mesh=mesh,
        compiler_params=pltpu.CompilerParams(needs_layout_passes=False),
        scratch_types=[
            pltpu.VMEM((K * NP,), F32),
            pltpu.VMEM((K * NP,), F32),
            pltpu.VMEM((NP,), I32),
            pltpu.VMEM((ECH,), I32),
            pltpu.VMEM((ECH,), I32),
        ],
    )
    def k(bv, srcl, dstl, cout, tbl, acc, lanescr, sbuf, dbuf):
        wid = lax.axis_index("s") * 2 + lax.axis_index("c")
        iota16 = lax.iota(I32, 16)
        for p in range(passes):
            slot = wid + p * 32

            @pl.when(slot < slots)
            def _():
                c0 = slot * K
                for kk in range(K):
                    pltpu.sync_copy(bv.at[c0 + kk],
                                    tbl.at[pl.ds(kk * NP, NP)])

                def init_body(j, _):
                    acc[pl.ds(j * 16, 16)] = jnp.full((16,), NEG, F32)
                    return 0

                lax.fori_loop(0, K * NP // 16, init_body, 0)

                def chunk_body(cidx, _):
                    pltpu.sync_copy(srcl.at[pl.ds(cidx * ECH, ECH)], sbuf)
                    pltpu.sync_copy(dstl.at[pl.ds(cidx * ECH, ECH)], dbuf)

                    def vec_body(v, _):
                        d = dbuf[pl.ds(v * 16, 16)]
                        s = sbuf[pl.ds(v * 16, 16)]
                        tvals = [plsc.load_gather(tbl, [s + (kk * NP)])
                                 for kk in range(K)]

                        def cond(act):
                            return jnp.max(act) > 0

                        def round_body(act):
                            actb = act > 0
                            plsc.store_scatter(lanescr, [d], iota16,
                                               mask=actb)
                            wv = plsc.load_gather(lanescr, [d])
                            win = jnp.logical_and(actb, wv == iota16)
                            for kk in range(K):
                                dk = d + (kk * NP)
                                av = plsc.load_gather(acc, [dk])
                                mx = jnp.maximum(av, tvals[kk])
                                plsc.store_scatter(acc, [dk], mx, mask=win)
                            return jnp.where(win, 0, act)

                        lax.while_loop(cond, round_body,
                                       jnp.ones((16,), I32))
                        return 0

                    lax.fori_loop(0, nvec, vec_body, 0)
                    return 0

                lax.fori_loop(0, nchunk, chunk_body, 0)
                for kk in range(K):
                    pltpu.sync_copy(acc.at[pl.ds(kk * NP, NP)],
                                    cout.at[c0 + kk])

    return k


def _split_conv_w(W, b):
    ci = W.shape[0] // 2
    Wi, Wj = W[:ci], W[ci:]
    WA = (Wi - Wj).T
    WB = Wj.T
    if ci == 6:
        WA = jnp.pad(WA, ((0, 0), (0, 2)))
        WB = jnp.pad(WB, ((0, 0), (0, 2)))
    return WA, WB, b[:, None]


def kernel(pos, x, tpl_edge_index, geo_edge_index, batch,
           Wt1, bt1, Wg1, bg1, Wm1, bm1,
           Wt2, bt2, Wg2, bg2, Wm2, bm2,
           Wt3, bt3, Wg3, bg3, Wm3, bm3,
           Wglb, bglb, Wtr, btr):
    N = pos.shape[0]
    E = tpl_edge_index.shape[1]

    xin = jnp.concatenate([pos, x], axis=1)
    xinT = jnp.zeros((8, NP), F32).at[:6, :N].set(xin.T)
    batchP = jnp.full((1, NP), 4, I32).at[0, :N].set(batch)
    pos128 = jnp.zeros((NP, 128), F32).at[:N, :3].set(pos)
    ts, td = tpl_edge_index[0], tpl_edge_index[1]
    gs, gd = geo_edge_index[0], geo_edge_index[1]

    WAt1, WBt1, bt1c = _split_conv_w(Wt1, bt1)
    WAg1, WBg1, bg1c = _split_conv_w(Wg1, bg1)
    WAt2, WBt2, bt2c = _split_conv_w(Wt2, bt2)
    WAg2, WBg2, bg2c = _split_conv_w(Wg2, bg2)
    WAt3, WBt3, bt3c = _split_conv_w(Wt3, bt3)
    WAg3, WBg3, bg3c = _split_conv_w(Wg3, bg3)
    Wm1T, Wm2T, Wm3T = Wm1.T, Wm2.T, Wm3.T
    bm1c, bm2c, bm3c = bm1[:, None], bm2[:, None], bm3[:, None]
    Wg1T = Wglb[:64].T
    Wg2T = Wglb[64:320].T
    Wg3T = Wglb[320:832].T
    bglbc = bglb[:, None]
    WtrT = Wtr.T
    Wtr_g = WtrT[:, :1024]
    Wtr_xin = jnp.pad(WtrT[:, 1024:1030], ((0, 0), (0, 2)))
    Wtr_1 = WtrT[:, 1030:1094]
    Wtr_2 = WtrT[:, 1094:1350]
    Wtr_3 = WtrT[:, 1350:1862]
    btrc = btr[:, None]

    At1, Bt1, Ag1, Bg1 = _t0(xinT, WAt1, WBt1, WAg1, WBg1, bt1c, bg1c)
    sc32 = _sc_segmax(32, 1, E)
    Ct1 = sc32(Bt1, ts, td)
    Cg1 = sc32(Bg1, gs, gd)

    x1, At2, Bt2, Ag2, Bg2 = _tmid(At1, Ct1, Ag1, Cg1, Wm1T, bm1c,
                                   WAt2, bt2c, WBt2, WAg2, bg2c, WBg2)
    sc128 = _sc_segmax(128, 4, E)
    Ct2 = sc128(Bt2, ts, td)
    Cg2 = sc128(Bg2, gs, gd)

    x2, At3, Bt3, Ag3, Bg3 = _tmid(At2, Ct2, Ag2, Cg2, Wm2T, bm2c,
                                   WAt3, bt3c, WBt3, WAg3, bg3c, WBg3)
    sc256 = _sc_segmax(256, 4, E)
    Ct3 = sc256(Bt3, ts, td)
    Cg3 = sc256(Bg3, gs, gd)

    x3, xgmax = _t3(At3, Ct3, Ag3, Cg3, Wm3T, bm3c, x1, x2,
                    Wg1T, Wg2T, Wg3T, bglbc, batchP)
    sigT, den = _t4(xgmax, batchP, xinT, x1, x2, x3,
                    Wtr_g, Wtr_xin, Wtr_1, Wtr_2, Wtr_3, btrc)
    wT, outsP = _t5(sigT, den, batchP, pos128)

    outs = outsP.reshape(4, 24, 128)[:, :, :3]
    w = wT[:, :N].T
    return outs, w


# speedup vs baseline: 1.9157x; 1.1355x over previous
import functools

import jax
import jax.numpy as jnp
from jax import lax
from jax.experimental import pallas as pl
from jax.experimental.pallas import tpu as pltpu
from jax.experimental.pallas import tpu_sc as plsc

F32 = jnp.float32
I32 = jnp.int32
NEG = -1e30
NP = 10240
BN = 512
NB = NP // BN
ECH = 2000


def _dot(a, b):
    return lax.dot_general(a, b, (((1,), (0,)), ((), ())),
                           preferred_element_type=F32)


def _nspec(c):
    return pl.BlockSpec((c, BN), lambda i: (0, i))


def _wspec(shape):
    return pl.BlockSpec(shape, lambda i: (0, 0))


def _t0_body(xin_ref, wat, wbt, wag, wbg, bt, bg, at_o, bt_o, ag_o, bg_o):
    X = xin_ref[...]
    at_o[...] = _dot(wat[...], X) + bt[...]
    bt_o[...] = _dot(wbt[...], X)
    ag_o[...] = _dot(wag[...], X) + bg[...]
    bg_o[...] = _dot(wbg[...], X)


def _t0(xinT, WAt, WBt, WAg, WBg, bt, bg):
    co = WAt.shape[0]
    return pl.pallas_call(
        _t0_body,
        grid=(NB,),
        in_specs=[_nspec(xinT.shape[0]), _wspec(WAt.shape), _wspec(WBt.shape),
                  _wspec(WAg.shape), _wspec(WBg.shape), _wspec(bt.shape),
                  _wspec(bg.shape)],
        out_specs=[_nspec(co)] * 4,
        out_shape=[jax.ShapeDtypeStruct((co, NP), F32)] * 4,
    )(xinT, WAt, WBt, WAg, WBg, bt, bg)


def _tmid_body(at, ct, ag, cg, wm, bm, wat, bt2, wbt, wag, bg2, wbg,
               x_o, a2t_o, b2t_o, a2g_o, b2g_o):
    h = jnp.concatenate([jnp.maximum(at[...] + ct[...], 0.0),
                         jnp.maximum(ag[...] + cg[...], 0.0)], axis=0)
    xn = jnp.maximum(_dot(wm[...], h) + bm[...], 0.0)
    x_o[...] = xn
    a2t_o[...] = _dot(wat[...], xn) + bt2[...]
    b2t_o[...] = _dot(wbt[...], xn)
    a2g_o[...] = _dot(wag[...], xn) + bg2[...]
    b2g_o[...] = _dot(wbg[...], xn)


def _tmid(at, ct, ag, cg, wm, bm, wat, bt2, wbt, wag, bg2, wbg):
    ci = at.shape[0]
    cx = wm.shape[0]
    co = wat.shape[0]
    return pl.pallas_call(
        _tmid_body,
        grid=(NB,),
        in_specs=[_nspec(ci)] * 4 + [_wspec(wm.shape), _wspec(bm.shape),
                  _wspec(wat.shape), _wspec(bt2.shape), _wspec(wbt.shape),
                  _wspec(wag.shape), _wspec(bg2.shape), _wspec(wbg.shape)],
        out_specs=[_nspec(cx)] + [_nspec(co)] * 4,
        out_shape=[jax.ShapeDtypeStruct((cx, NP), F32)] +
                  [jax.ShapeDtypeStruct((co, NP), F32)] * 4,
    )(at, ct, ag, cg, wm, bm, wat, bt2, wbt, wag, bg2, wbg)


def _t3_body(at, ct, ag, cg, wm3, bm3, x1, x2, wg1, wg2, wg3, bglb, bb_ref,
             x3_o, xg_o):
    i = pl.program_id(0)
    h = jnp.concatenate([jnp.maximum(at[...] + ct[...], 0.0),
                         jnp.maximum(ag[...] + cg[...], 0.0)], axis=0)
    x3 = jnp.maximum(_dot(wm3[...], h) + bm3[...], 0.0)
    x3_o[...] = x3
    x4 = jnp.maximum(_dot(wg1[...], x1[...]) + _dot(wg2[...], x2[...]) +
                     _dot(wg3[...], x3) + bglb[...], 0.0)

    @pl.when(i == 0)
    def _():
        xg_o[...] = jnp.full((1024, 128), NEG, F32)

    bb = bb_ref[...]
    lane = lax.broadcasted_iota(I32, (1024, 128), 1)
    acc = xg_o[...]
    for b in range(4):
        colmax = jnp.max(jnp.where(bb == b, x4, NEG), axis=1, keepdims=True)
        acc = jnp.maximum(acc, jnp.where(lane == b, colmax, NEG))
    xg_o[...] = acc


def _t3(at, ct, ag, cg, wm3, bm3, x1, x2, wg1, wg2, wg3, bglb, batchP):
    return pl.pallas_call(
        _t3_body,
        grid=(NB,),
        in_specs=[_nspec(256)] * 4 + [_wspec(wm3.shape), _wspec(bm3.shape),
                  _nspec(64), _nspec(256), _wspec(wg1.shape),
                  _wspec(wg2.shape), _wspec(wg3.shape), _wspec(bglb.shape),
                  _nspec(1)],
        out_specs=[_nspec(512), pl.BlockSpec((1024, 128), lambda i: (0, 0))],
        out_shape=[jax.ShapeDtypeStruct((512, NP), F32),
                   jax.ShapeDtypeStruct((1024, 128), F32)],
    )(at, ct, ag, cg, wm3, bm3, x1, x2, wg1, wg2, wg3, bglb, batchP)


def _t4_body(xgm_ref, bb_ref, xin, x1, x2, x3, wg, wxin, w1, w2, w3, btr,
             sig_o, den_o):
    i = pl.program_id(0)
    xgm = xgm_ref[...]
    xg = jnp.where(xgm > -1e29, xgm, 0.0)
    bb = bb_ref[...]
    xgb = jnp.zeros((1024, BN), F32)
    for b in range(4):
        xgb = xgb + jnp.where(bb == b, xg[:, b:b + 1], 0.0)
    z = (_dot(wg[...], xgb) + _dot(wxin[...], xin[...]) +
         _dot(w1[...], x1[...]) + _dot(w2[...], x2[...]) +
         _dot(w3[...], x3[...]) + btr[...])
    sig = 1.0 / (1.0 + jnp.exp(-z))
    sig_o[...] = sig

    @pl.when(i == 0)
    def _():
        den_o[...] = jnp.zeros((24, 128), F32)

    lane = lax.broadcasted_iota(I32, (24, 128), 1)
    acc = den_o[...]
    for b in range(4):
        colsum = jnp.sum(jnp.where(bb == b, sig, 0.0), axis=1, keepdims=True)
        acc = acc + jnp.where(lane == b, colsum, 0.0)
    den_o[...] = acc


def _t4(xgmax, batchP, xinT, x1, x2, x3, wg, wxin, w1, w2, w3, btr):
    return pl.pallas_call(
        _t4_body,
        grid=(NB,),
        in_specs=[pl.BlockSpec((1024, 128), lambda i: (0, 0)), _nspec(1),
                  _nspec(8), _nspec(64), _nspec(256), _nspec(512),
                  _wspec(wg.shape), _wspec(wxin.shape), _wspec(w1.shape),
                  _wspec(w2.shape), _wspec(w3.shape), _wspec(btr.shape)],
        out_specs=[_nspec(24), pl.BlockSpec((24, 128), lambda i: (0, 0))],
        out_shape=[jax.ShapeDtypeStruct((24, NP), F32),
                   jax.ShapeDtypeStruct((24, 128), F32)],
    )(xgmax, batchP, xinT, x1, x2, x3, wg, wxin, w1, w2, w3, btr)


def _t5_body(sig_ref, den_ref, bb_ref, pos_ref, w_o, outs_o):
    i = pl.program_id(0)
    den = den_ref[...]
    recip = 1.0 / (den + 1e-9)
    bb = bb_ref[...]
    rsel = jnp.zeros((24, BN), F32)
    for b in range(4):
        rsel = rsel + jnp.where(bb == b, recip[:, b:b + 1], 0.0)
    w = sig_ref[...] * rsel
    w_o[...] = w

    @pl.when(i == 0)
    def _():
        outs_o[...] = jnp.zeros((96, 128), F32)

    acc = outs_o[...]
    pos = pos_ref[...]
    parts = []
    for b in range(4):
        mw = jnp.where(bb == b, w, 0.0)
        parts.append(_dot(mw, pos))
    outs_o[...] = acc + jnp.concatenate(parts, axis=0)


def _t5(sigT, den, batchP, pos128):
    return pl.pallas_call(
        _t5_body,
        grid=(NB,),
        in_specs=[_nspec(24), pl.BlockSpec((24, 128), lambda i: (0, 0)),
                  _nspec(1), pl.BlockSpec((BN, 128), lambda i: (i, 0))],
        out_specs=[_nspec(24), pl.BlockSpec((96, 128), lambda i: (0, 0))],
        out_shape=[jax.ShapeDtypeStruct((24, NP), F32),
                   jax.ShapeDtypeStruct((96, 128), F32)],
    )(sigT, den, batchP, pos128)


def _sc_segmax(ch, K, E):
    slots = ch // K
    passes = -(-slots // 32)
    nvec = ECH // 16
    nchunk = E // ECH
    mesh = plsc.VectorSubcoreMesh(core_axis_name="c", subcore_axis_name="s")

    @functools.partial(
        pl.kernel,
        out_type=jax.ShapeDtypeStruct((ch, NP), F32),
        mesh=mesh,
        compiler_params=pltpu.CompilerParams(needs_layout_passes=False),
        scratch_types=[
            pltpu.VMEM((K * NP,), F32),
            pltpu.VMEM((K * NP,), F32),
            pltpu.VMEM((NP,), I32),
            pltpu.VMEM((2 * ECH,), I32),
            pltpu.VMEM((2 * ECH,), I32),
            pltpu.SemaphoreType.DMA,
            pltpu.SemaphoreType.DMA,
        ],
    )
    def k(bv, srcl, dstl, cout, tbl, acc, lanescr, sring, dring, sem0, sem1):
        wid = lax.axis_index("s") * 2 + lax.axis_index("c")
        iota16 = lax.iota(I32, 16)
        sems = [sem0, sem1]

        def start(cidx, b):
            pltpu.async_copy(srcl.at[pl.ds(cidx * ECH, ECH)],
                             sring.at[pl.ds(b * ECH, ECH)], sems[b])
            pltpu.async_copy(dstl.at[pl.ds(cidx * ECH, ECH)],
                             dring.at[pl.ds(b * ECH, ECH)], sems[b])

        def wait(cidx, b):
            pltpu.make_async_copy(srcl.at[pl.ds(cidx * ECH, ECH)],
                                  sring.at[pl.ds(b * ECH, ECH)],
                                  sems[b]).wait()
            pltpu.make_async_copy(dstl.at[pl.ds(cidx * ECH, ECH)],
                                  dring.at[pl.ds(b * ECH, ECH)],
                                  sems[b]).wait()

        def process_chunk(b):
            def vec_body(v, _):
                d = dring[pl.ds(b * ECH + v * 16, 16)]
                s = sring[pl.ds(b * ECH + v * 16, 16)]
                tvals = [plsc.load_gather(tbl, [s + (kk * NP)])
                         for kk in range(K)]
                plsc.store_scatter(lanescr, [d], iota16)
                wv = plsc.load_gather(lanescr, [d])
                win = wv == iota16
                for kk in range(K):
                    dk = d + (kk * NP)
                    av = plsc.load_gather(acc, [dk])
                    mx = jnp.maximum(av, tvals[kk])
                    plsc.store_scatter(acc, [dk], mx, mask=win)
                losers = jnp.logical_not(win)
                cnt = plsc.all_reduce_population_count(losers)

                @pl.when(cnt[0] > 0)
                def _():
                    def cond(act):
                        return jnp.max(act) > 0

                    def round_body(act):
                        actb = act > 0
                        plsc.store_scatter(lanescr, [d], iota16, mask=actb)
                        wv2 = plsc.load_gather(lanescr, [d])
                        win2 = jnp.logical_and(actb, wv2 == iota16)
                        for kk in range(K):
                            dk = d + (kk * NP)
                            av = plsc.load_gather(acc, [dk])
                            mx = jnp.maximum(av, tvals[kk])
                            plsc.store_scatter(acc, [dk], mx, mask=win2)
                        return jnp.where(win2, 0, act)

                    lax.while_loop(cond, round_body, losers.astype(I32))

                return 0

            lax.fori_loop(0, nvec, vec_body, 0)

        for p in range(passes):
            slot = wid + p * 32

            @pl.when(slot < slots)
            def _():
                c0 = slot * K
                for kk in range(K):
                    pltpu.sync_copy(bv.at[c0 + kk],
                                    tbl.at[pl.ds(kk * NP, NP)])

                def init_body(j, _):
                    acc[pl.ds(j * 16, 16)] = jnp.full((16,), NEG, F32)
                    return 0

                lax.fori_loop(0, K * NP // 16, init_body, 0)

                start(0, 0)

                def pair_body(g, _):
                    for b in range(2):
                        cidx = g * 2 + b
                        wait(cidx, b)

                        @pl.when(cidx + 1 < nchunk)
                        def _():
                            start(cidx + 1, 1 - b)

                        process_chunk(b)
                    return 0

                lax.fori_loop(0, nchunk // 2, pair_body, 0)
                for kk in range(K):
                    pltpu.sync_copy(acc.at[pl.ds(kk * NP, NP)],
                                    cout.at[c0 + kk])

    return k


def _split_conv_w(W, b):
    ci = W.shape[0] // 2
    Wi, Wj = W[:ci], W[ci:]
    WA = (Wi - Wj).T
    WB = Wj.T
    if ci == 6:
        WA = jnp.pad(WA, ((0, 0), (0, 2)))
        WB = jnp.pad(WB, ((0, 0), (0, 2)))
    return WA, WB, b[:, None]


def kernel(pos, x, tpl_edge_index, geo_edge_index, batch,
           Wt1, bt1, Wg1, bg1, Wm1, bm1,
           Wt2, bt2, Wg2, bg2, Wm2, bm2,
           Wt3, bt3, Wg3, bg3, Wm3, bm3,
           Wglb, bglb, Wtr, btr):
    N = pos.shape[0]
    E = tpl_edge_index.shape[1]

    xin = jnp.concatenate([pos, x], axis=1)
    xinT = jnp.zeros((8, NP), F32).at[:6, :N].set(xin.T)
    batchP = jnp.full((1, NP), 4, I32).at[0, :N].set(batch)
    pos128 = jnp.zeros((NP, 128), F32).at[:N, :3].set(pos)
    ts, td = tpl_edge_index[0], tpl_edge_index[1]
    gs, gd = geo_edge_index[0], geo_edge_index[1]

    WAt1, WBt1, bt1c = _split_conv_w(Wt1, bt1)
    WAg1, WBg1, bg1c = _split_conv_w(Wg1, bg1)
    WAt2, WBt2, bt2c = _split_conv_w(Wt2, bt2)
    WAg2, WBg2, bg2c = _split_conv_w(Wg2, bg2)
    WAt3, WBt3, bt3c = _split_conv_w(Wt3, bt3)
    WAg3, WBg3, bg3c = _split_conv_w(Wg3, bg3)
    Wm1T, Wm2T, Wm3T = Wm1.T, Wm2.T, Wm3.T
    bm1c, bm2c, bm3c = bm1[:, None], bm2[:, None], bm3[:, None]
    Wg1T = Wglb[:64].T
    Wg2T = Wglb[64:320].T
    Wg3T = Wglb[320:832].T
    bglbc = bglb[:, None]
    WtrT = Wtr.T
    Wtr_g = WtrT[:, :1024]
    Wtr_xin = jnp.pad(WtrT[:, 1024:1030], ((0, 0), (0, 2)))
    Wtr_1 = WtrT[:, 1030:1094]
    Wtr_2 = WtrT[:, 1094:1350]
    Wtr_3 = WtrT[:, 1350:1862]
    btrc = btr[:, None]

    At1, Bt1, Ag1, Bg1 = _t0(xinT, WAt1, WBt1, WAg1, WBg1, bt1c, bg1c)
    sc32 = _sc_segmax(32, 1, E)
    Ct1 = sc32(Bt1, ts, td)
    Cg1 = sc32(Bg1, gs, gd)

    x1, At2, Bt2, Ag2, Bg2 = _tmid(At1, Ct1, Ag1, Cg1, Wm1T, bm1c,
                                   WAt2, bt2c, WBt2, WAg2, bg2c, WBg2)
    sc128 = _sc_segmax(128, 4, E)
    Ct2 = sc128(Bt2, ts, td)
    Cg2 = sc128(Bg2, gs, gd)

    x2, At3, Bt3, Ag3, Bg3 = _tmid(At2, Ct2, Ag2, Cg2, Wm2T, bm2c,
                                   WAt3, bt3c, WBt3, WAg3, bg3c, WBg3)
    sc256 = _sc_segmax(256, 4, E)
    Ct3 = sc256(Bt3, ts, td)
    Cg3 = sc256(Bg3, gs, gd)

    x3, xgmax = _t3(At3, Ct3, Ag3, Cg3, Wm3T, bm3c, x1, x2,
                    Wg1T, Wg2T, Wg3T, bglbc, batchP)
    sigT, den = _t4(xgmax, batchP, xinT, x1, x2, x3,
                    Wtr_g, Wtr_xin, Wtr_1, Wtr_2, Wtr_3, btrc)
    wT, outsP = _t5(sigT, den, batchP, pos128)

    outs = outsP.reshape(4, 24, 128)[:, :, :3]
    w = wT[:, :N].T
    return outs, w


# speedup vs baseline: 7.6371x; 3.9865x over previous
import functools

import jax
import jax.numpy as jnp
from jax import lax
from jax.experimental import pallas as pl
from jax.experimental.pallas import tpu as pltpu
from jax.experimental.pallas import tpu_sc as plsc

F32 = jnp.float32
I32 = jnp.int32
NEG = -1e30
NP = 10240
BN = 512
NB = NP // BN
ECH = 2000
FL = 2048
LB = 6144
GB = 64
ROWS = 328


def _sds(shape, dt):
    return jax.ShapeDtypeStruct(shape, dt)


def _nspec(c):
    return pl.BlockSpec((c, BN), lambda i: (0, i))


def _mspec(c):
    return pl.BlockSpec((BN, c), lambda i: (i, 0))


def _wspec(shape):
    return pl.BlockSpec(shape, lambda i: (0, 0))


def _d00(a, b):
    return lax.dot_general(a, b, (((0,), (0,)), ((), ())),
                           preferred_element_type=F32)


def _d01(a, b):
    return lax.dot_general(a, b, (((0,), (1,)), ((), ())),
                           preferred_element_type=F32)


def _d10(a, b):
    return lax.dot_general(a, b, (((1,), (0,)), ((), ())),
                           preferred_element_type=F32)


def _t0_body(xin_ref, wat, wbt, wag, wbg, bt, bg, at_o, bt_o, ag_o, bg_o):
    X = xin_ref[...]
    at_o[...] = _d00(X, wat[...]) + bt[...]
    bt_o[...] = _d00(X, wbt[...])
    ag_o[...] = _d00(X, wag[...]) + bg[...]
    bg_o[...] = _d00(X, wbg[...])


def _t0(xinT, WAt, WBt, WAg, WBg, bt, bg):
    co = WAt.shape[1]
    return pl.pallas_call(
        _t0_body,
        grid=(NB,),
        in_specs=[_nspec(xinT.shape[0]), _wspec(WAt.shape), _wspec(WBt.shape),
                  _wspec(WAg.shape), _wspec(WBg.shape), _wspec(bt.shape),
                  _wspec(bg.shape)],
        out_specs=[_mspec(co)] * 4,
        out_shape=[_sds((NP, co), F32)] * 4,
    )(xinT, WAt, WBt, WAg, WBg, bt, bg)


def _tmid_body(at, ct, ag, cg, wm, bm, wat, bt2, wbt, wag, bg2, wbg,
               x_o, a2t_o, b2t_o, a2g_o, b2g_o):
    h = jnp.concatenate([jnp.maximum(at[...] + ct[...], 0.0),
                         jnp.maximum(ag[...] + cg[...], 0.0)], axis=1)
    xn = jnp.maximum(_d01(wm[...], h) + bm[...], 0.0)
    x_o[...] = xn
    a2t_o[...] = _d00(xn, wat[...]) + bt2[...]
    b2t_o[...] = _d00(xn, wbt[...])
    a2g_o[...] = _d00(xn, wag[...]) + bg2[...]
    b2g_o[...] = _d00(xn, wbg[...])


def _tmid(at, ct, ag, cg, wm, bm, wat, bt2, wbt, wag, bg2, wbg):
    ci = at.shape[1]
    cx = wm.shape[1]
    co = wat.shape[1]
    return pl.pallas_call(
        _tmid_body,
        grid=(NB,),
        in_specs=[_mspec(ci)] * 4 + [_wspec(wm.shape), _wspec(bm.shape),
                  _wspec(wat.shape), _wspec(bt2.shape), _wspec(wbt.shape),
                  _wspec(wag.shape), _wspec(bg2.shape), _wspec(wbg.shape)],
        out_specs=[_nspec(cx)] + [_mspec(co)] * 4,
        out_shape=[_sds((cx, NP), F32)] + [_sds((NP, co), F32)] * 4,
    )(at, ct, ag, cg, wm, bm, wat, bt2, wbt, wag, bg2, wbg)


def _t3_body(at, ct, ag, cg, wm3, bm3, x1, x2, wg1, wg2, wg3, bglb, bb_ref,
             x3_o, xg_o):
    i = pl.program_id(0)
    h = jnp.concatenate([jnp.maximum(at[...] + ct[...], 0.0),
                         jnp.maximum(ag[...] + cg[...], 0.0)], axis=1)
    x3 = jnp.maximum(_d01(wm3[...], h) + bm3[...], 0.0)
    x3_o[...] = x3
    x4 = jnp.maximum(_d10(wg1[...], x1[...]) + _d10(wg2[...], x2[...]) +
                     _d10(wg3[...], x3) + bglb[...], 0.0)

    @pl.when(i == 0)
    def _():
        xg_o[...] = jnp.full((1024, 128), NEG, F32)

    bb = bb_ref[...]
    lane = lax.broadcasted_iota(I32, (1024, 128), 1)
    acc = xg_o[...]
    for b in range(4):
        colmax = jnp.max(jnp.where(bb == b, x4, NEG), axis=1, keepdims=True)
        acc = jnp.maximum(acc, jnp.where(lane == b, colmax, NEG))
    xg_o[...] = acc


def _t3(at, ct, ag, cg, wm3, bm3, x1, x2, wg1, wg2, wg3, bglb, batchP):
    return pl.pallas_call(
        _t3_body,
        grid=(NB,),
        in_specs=[_mspec(256)] * 4 + [_wspec(wm3.shape), _wspec(bm3.shape),
                  _nspec(64), _nspec(256), _wspec(wg1.shape),
                  _wspec(wg2.shape), _wspec(wg3.shape), _wspec(bglb.shape),
                  _nspec(1)],
        out_specs=[_nspec(512), pl.BlockSpec((1024, 128), lambda i: (0, 0))],
        out_shape=[_sds((512, NP), F32), _sds((1024, 128), F32)],
    )(at, ct, ag, cg, wm3, bm3, x1, x2, wg1, wg2, wg3, bglb, batchP)


def _t4_body(xgm_ref, bb_ref, xin, x1, x2, x3, wg, wxin, w1, w2, w3, btr,
             sig_o, den_o):
    i = pl.program_id(0)
    xgm = xgm_ref[...]
    xg = jnp.where(xgm > -1e29, xgm, 0.0)
    bb = bb_ref[...]
    xgb = jnp.zeros((1024, BN), F32)
    for b in range(4):
        xgb = xgb + jnp.where(bb == b, xg[:, b:b + 1], 0.0)
    z = (_d10(wg[...], xgb) + _d10(wxin[...], xin[...]) +
         _d10(w1[...], x1[...]) + _d10(w2[...], x2[...]) +
         _d10(w3[...], x3[...]) + btr[...])
    sig = 1.0 / (1.0 + jnp.exp(-z))
    sig_o[...] = sig

    @pl.when(i == 0)
    def _():
        den_o[...] = jnp.zeros((24, 128), F32)

    lane = lax.broadcasted_iota(I32, (24, 128), 1)
    acc = den_o[...]
    for b in range(4):
        colsum = jnp.sum(jnp.where(bb == b, sig, 0.0), axis=1, keepdims=True)
        acc = acc + jnp.where(lane == b, colsum, 0.0)
    den_o[...] = acc


def _t4(xgmax, batchP, xinT, x1, x2, x3, wg, wxin, w1, w2, w3, btr):
    return pl.pallas_call(
        _t4_body,
        grid=(NB,),
        in_specs=[pl.BlockSpec((1024, 128), lambda i: (0, 0)), _nspec(1),
                  _nspec(8), _nspec(64), _nspec(256), _nspec(512),
                  _wspec(wg.shape), _wspec(wxin.shape), _wspec(w1.shape),
                  _wspec(w2.shape), _wspec(w3.shape), _wspec(btr.shape)],
        out_specs=[_nspec(24), pl.BlockSpec((24, 128), lambda i: (0, 0))],
        out_shape=[_sds((24, NP), F32), _sds((24, 128), F32)],
    )(xgmax, batchP, xinT, x1, x2, x3, wg, wxin, w1, w2, w3, btr)


def _t5_body(sig_ref, den_ref, bb_ref, pos_ref, w_o, outs_o):
    i = pl.program_id(0)
    den = den_ref[...]
    recip = 1.0 / (den + 1e-9)
    bb = bb_ref[...]
    rsel = jnp.zeros((24, BN), F32)
    for b in range(4):
        rsel = rsel + jnp.where(bb == b, recip[:, b:b + 1], 0.0)
    w = sig_ref[...] * rsel
    w_o[...] = w

    @pl.when(i == 0)
    def _():
        outs_o[...] = jnp.zeros((96, 128), F32)

    acc = outs_o[...]
    pos = pos_ref[...]
    parts = []
    for b in range(4):
        mw = jnp.where(bb == b, w, 0.0)
        parts.append(_d10(mw, pos))
    outs_o[...] = acc + jnp.concatenate(parts, axis=0)


def _t5(sigT, den, batchP, pos128):
    return pl.pallas_call(
        _t5_body,
        grid=(NB,),
        in_specs=[_nspec(24), pl.BlockSpec((24, 128), lambda i: (0, 0)),
                  _nspec(1), pl.BlockSpec((BN, 128), lambda i: (i, 0))],
        out_specs=[_nspec(24), pl.BlockSpec((96, 128), lambda i: (0, 0))],
        out_shape=[_sds((24, NP), F32), _sds((96, 128), F32)],
    )(sigT, den, batchP, pos128)


def _sc_partition(E):
    CAPP = E + 2 * FL
    nchunk = E // ECH
    mesh = plsc.VectorSubcoreMesh(core_axis_name="c", subcore_axis_name="s")

    @functools.partial(
        pl.kernel,
        out_type=[_sds((32 * CAPP,), I32), _sds((32, 16), I32)],
        mesh=mesh,
        compiler_params=pltpu.CompilerParams(needs_layout_passes=False),
        scratch_types=[
            pltpu.VMEM((LB,), I32),
            pltpu.VMEM((2 * ECH,), I32),
            pltpu.VMEM((2 * ECH,), I32),
            pltpu.VMEM((16,), I32),
            pltpu.SMEM((2,), I32),
            pltpu.SemaphoreType.DMA,
            pltpu.SemaphoreType.DMA,
        ],
    )
    def k(srcl, dstl, pk_out, cnt_out, lbuf, sring, dring, cbuf, sref,
          sem0, sem1):
        wid = lax.axis_index("s") * 2 + lax.axis_index("c")
        lo = wid * 320
        base = wid * CAPP
        iota16 = lax.iota(I32, 16)
        sems = [sem0, sem1]

        def start(cidx, b):
            pltpu.async_copy(srcl.at[pl.ds(cidx * ECH, ECH)],
                             sring.at[pl.ds(b * ECH, ECH)], sems[b])
            pltpu.async_copy(dstl.at[pl.ds(cidx * ECH, ECH)],
                             dring.at[pl.ds(b * ECH, ECH)], sems[b])

        def wait(cidx, b):
            pltpu.make_async_copy(srcl.at[pl.ds(cidx * ECH, ECH)],
                                  sring.at[pl.ds(b * ECH, ECH)],
                                  sems[b]).wait()
            pltpu.make_async_copy(dstl.at[pl.ds(cidx * ECH, ECH)],
                                  dring.at[pl.ds(b * ECH, ECH)],
                                  sems[b]).wait()

        sref[0] = 0
        sref[1] = 0
        start(0, 0)

        def pair_body(g, _):
            for b in range(2):
                cidx = g * 2 + b
                wait(cidx, b)

                @pl.when(cidx + 1 < nchunk)
                def _():
                    start(cidx + 1, 1 - b)

                def vec_body(v, _):
                    d = dring[pl.ds(b * ECH + v * 16, 16)]
                    s = sring[pl.ds(b * ECH + v * 16, 16)]
                    m = jnp.logical_and(d >= lo, d < lo + 320)
                    pk = s * 512 + (d - lo)
                    cnt = sref[0]
                    pref = plsc.cumsum(m.astype(I32))
                    plsc.store_scatter(lbuf, [cnt + pref - 1], pk, mask=m)
                    sref[0] = cnt + pref[15]
                    return 0

                lax.fori_loop(0, ECH // 16, vec_body, 0)

                @pl.when(sref[0] >= FL)
                def _():
                    off = pl.multiple_of(base + sref[1], 8)
                    pltpu.sync_copy(lbuf.at[pl.ds(0, FL)],
                                    pk_out.at[pl.ds(off, FL)])

                    def mv(j, _):
                        lbuf[pl.ds(j * 16, 16)] = lbuf[pl.ds(FL + j * 16, 16)]
                        return 0

                    lax.fori_loop(0, (LB - FL) // 16, mv, 0)
                    sref[0] = sref[0] - FL
                    sref[1] = sref[1] + FL
            return 0

        lax.fori_loop(0, nchunk // 2, pair_body, 0)

        cnt = sref[0]

        def dum(j, _):
            plsc.store_scatter(lbuf, [cnt + iota16 + j * 16],
                               jnp.full((16,), 320, I32))
            return 0

        lax.fori_loop(0, FL // 16, dum, 0)
        off = pl.multiple_of(base + sref[1], 8)
        pltpu.sync_copy(lbuf.at[pl.ds(0, FL)],
                        pk_out.at[pl.ds(off, FL)])

        @pl.when(cnt > FL)
        def _():
            pltpu.sync_copy(lbuf.at[pl.ds(FL, FL)],
                            pk_out.at[pl.ds(off + FL, FL)])

        cntp = ((cnt + FL - 1) // FL) * FL
        cbuf[...] = jnp.full((16,), cntp, I32)
        pltpu.sync_copy(cbuf, cnt_out.at[wid])

    return k


def _sc_agg(ch, E):
    CAPP = E + 2 * FL
    CH16 = ch // 16
    mesh = plsc.VectorSubcoreMesh(core_axis_name="c", subcore_axis_name="s")

    @functools.partial(
        pl.kernel,
        out_type=_sds((NP, ch), F32),
        mesh=mesh,
        compiler_params=pltpu.CompilerParams(needs_layout_passes=False,
                                             use_tc_tiling_on_sc=False),
        scratch_types=[
            pltpu.VMEM((ROWS, ch), F32),
            pltpu.VMEM((GB, ch), F32),
            pltpu.VMEM((GB, ch), F32),
            pltpu.VMEM((GB,), I32),
            pltpu.VMEM((GB,), I32),
            pltpu.VMEM((GB,), I32),
            pltpu.VMEM((GB,), I32),
            pltpu.VMEM((GB,), I32),
            pltpu.VMEM((GB,), I32),
            pltpu.VMEM((16,), I32),
            pltpu.SemaphoreType.DMA,
            pltpu.SemaphoreType.DMA,
            pltpu.SemaphoreType.DMA,
            pltpu.SemaphoreType.DMA,
        ],
    )
    def k(pkl, cntl, tblh, cout, acc, rows0, rows1, pk0, pk1, idx0, idx1,
          dl0, dl1, cbuf, spk0, spk1, sg0, sg1):
        wid = lax.axis_index("s") * 2 + lax.axis_index("c")
        lo = wid * 320
        base = wid * CAPP
        rows = [rows0, rows1]
        pkb = [pk0, pk1]
        idxb = [idx0, idx1]
        dlb = [dl0, dl1]
        spk = [spk0, spk1]
        sg = [sg0, sg1]

        pltpu.sync_copy(cntl.at[wid], cbuf)
        cntp = cbuf[...][0]
        nb = cntp // GB

        def initb(r, _):
            for c in range(CH16):
                acc[r, pl.ds(c * 16, 16)] = jnp.full((16,), NEG, F32)
            return 0

        lax.fori_loop(0, ROWS, initb, 0)

        def pk_start(i, b):
            o = pl.multiple_of(base + i * GB, 8)
            pltpu.async_copy(pkl.at[pl.ds(o, GB)], pkb[b], spk[b])

        def pk_wait(i, b):
            o = pl.multiple_of(base + i * GB, 8)
            pltpu.make_async_copy(pkl.at[pl.ds(o, GB)], pkb[b],
                                  spk[b]).wait()

        def build(b):
            for v in range(GB // 16):
                p = pkb[b][pl.ds(v * 16, 16)]
                idxb[b][pl.ds(v * 16, 16)] = lax.shift_right_logical(p, 9)
                dlb[b][pl.ds(v * 16, 16)] = lax.bitwise_and(p, 511)

        def g_start(b):
            pltpu.async_copy(tblh.at[idxb[b]], rows[b], sg[b])

        def g_wait(b):
            pltpu.make_async_copy(tblh.at[idxb[b]], rows[b], sg[b]).wait()

        @pl.when(nb > 0)
        def _():
            pk_start(0, 0)
            pk_wait(0, 0)
            build(0)
            g_start(0)

            @pl.when(nb > 1)
            def _():
                pk_start(1, 1)

            def pair_body(g2, _):
                for b in range(2):
                    i = g2 * 2 + b

                    @pl.when(i + 1 < nb)
                    def _():
                        pk_wait(i + 1, 1 - b)
                        build(1 - b)
                        g_start(1 - b)

                    @pl.when(i + 2 < nb)
                    def _():
                        pk_start(i + 2, b)

                    g_wait(b)

                    def vgroup(vv, _):
                        dlv = dlb[b][pl.ds(vv * 16, 16)]
                        for j in range(16):
                            dloc = dlv[j]
                            e = vv * 16 + j
                            for c in range(CH16):
                                r = rows[b][e, pl.ds(c * 16, 16)]
                                av = acc[dloc, pl.ds(c * 16, 16)]
                                acc[dloc, pl.ds(c * 16, 16)] = (
                                    jnp.maximum(av, r))
                        return 0

                    lax.fori_loop(0, GB // 16, vgroup, 0)
                return 0

            lax.fori_loop(0, nb // 2, pair_body, 0)

        pltpu.sync_copy(acc.at[pl.ds(0, 320)], cout.at[pl.ds(lo, 320)])

    return k


def _split_conv_w(W, b):
    ci = W.shape[0] // 2
    Wi, Wj = W[:ci], W[ci:]
    WA = Wi - Wj
    WB = Wj
    if ci == 6:
        WA = jnp.pad(WA, ((0, 2), (0, 0)))
        WB = jnp.pad(WB, ((0, 2), (0, 0)))
    return WA, WB, b[None, :]


def kernel(pos, x, tpl_edge_index, geo_edge_index, batch,
           Wt1, bt1, Wg1, bg1, Wm1, bm1,
           Wt2, bt2, Wg2, bg2, Wm2, bm2,
           Wt3, bt3, Wg3, bg3, Wm3, bm3,
           Wglb, bglb, Wtr, btr):
    N = pos.shape[0]
    E = tpl_edge_index.shape[1]

    xin = jnp.concatenate([pos, x], axis=1)
    xinT = jnp.zeros((8, NP), F32).at[:6, :N].set(xin.T)
    batchP = jnp.full((1, NP), 4, I32).at[0, :N].set(batch)
    pos128 = jnp.zeros((NP, 128), F32).at[:N, :3].set(pos)
    ts, td = tpl_edge_index[0], tpl_edge_index[1]
    gs, gd = geo_edge_index[0], geo_edge_index[1]

    WAt1, WBt1, bt1c = _split_conv_w(Wt1, bt1)
    WAg1, WBg1, bg1c = _split_conv_w(Wg1, bg1)
    WAt2, WBt2, bt2c = _split_conv_w(Wt2, bt2)
    WAg2, WBg2, bg2c = _split_conv_w(Wg2, bg2)
    WAt3, WBt3, bt3c = _split_conv_w(Wt3, bt3)
    WAg3, WBg3, bg3c = _split_conv_w(Wg3, bg3)
    bm1c, bm2c, bm3c = bm1[:, None], bm2[:, None], bm3[:, None]
    Wg1T = Wglb[:64].T
    Wg2T = Wglb[64:320].T
    Wg3T = Wglb[320:832].T
    bglbc = bglb[:, None]
    WtrT = Wtr.T
    Wtr_g = WtrT[:, :1024]
    Wtr_xin = jnp.pad(WtrT[:, 1024:1030], ((0, 0), (0, 2)))
    Wtr_1 = WtrT[:, 1030:1094]
    Wtr_2 = WtrT[:, 1094:1350]
    Wtr_3 = WtrT[:, 1350:1862]
    btrc = btr[:, None]

    part = _sc_partition(E)
    pk_t, cn_t = part(ts, td)
    pk_g, cn_g = part(gs, gd)
    agg32 = _sc_agg(32, E)
    agg128 = _sc_agg(128, E)
    agg256 = _sc_agg(256, E)

    At1, Bt1, Ag1, Bg1 = _t0(xinT, WAt1, WBt1, WAg1, WBg1, bt1c, bg1c)
    Ct1 = agg32(pk_t, cn_t, Bt1)
    Cg1 = agg32(pk_g, cn_g, Bg1)

    x1, At2, Bt2, Ag2, Bg2 = _tmid(At1, Ct1, Ag1, Cg1, Wm1, bm1c,
                                   WAt2, bt2c, WBt2, WAg2, bg2c, WBg2)
    Ct2 = agg128(pk_t, cn_t, Bt2)
    Cg2 = agg128(pk_g, cn_g, Bg2)

    x2, At3, Bt3, Ag3, Bg3 = _tmid(At2, Ct2, Ag2, Cg2, Wm2, bm2c,
                                   WAt3, bt3c, WBt3, WAg3, bg3c, WBg3)
    Ct3 = agg256(pk_t, cn_t, Bt3)
    Cg3 = agg256(pk_g, cn_g, Bg3)

    x3, xgmax = _t3(At3, Ct3, Ag3, Cg3, Wm3, bm3c, x1, x2,
                    Wg1T, Wg2T, Wg3T, bglbc, batchP)
    sigT, den = _t4(xgmax, batchP, xinT, x1, x2, x3,
                    Wtr_g, Wtr_xin, Wtr_1, Wtr_2, Wtr_3, btrc)
    wT, outsP = _t5(sigT, den, batchP, pos128)

    outs = outsP.reshape(4, 24, 128)[:, :, :3]
    w = wT[:, :N].T
    return outs, w


# speedup vs baseline: 9.5160x; 1.2460x over previous
import functools

import jax
import jax.numpy as jnp
from jax import lax
from jax.experimental import pallas as pl
from jax.experimental.pallas import tpu as pltpu
from jax.experimental.pallas import tpu_sc as plsc

F32 = jnp.float32
BF16 = jnp.bfloat16
I32 = jnp.int32
NEG = -1e30
NP = 10240
BN = 512
NB = NP // BN
ECH = 2000
FL = 2048
LB = 6144
GB = 128
ROWS = 328


def _sds(shape, dt):
    return jax.ShapeDtypeStruct(shape, dt)


def _nspec(c):
    return pl.BlockSpec((c, BN), lambda i: (0, i))


def _mspec(c):
    return pl.BlockSpec((BN, c), lambda i: (i, 0))


def _wspec(shape):
    return pl.BlockSpec(shape, lambda i: (0, 0))


def _d00(a, b):
    return lax.dot_general(a, b, (((0,), (0,)), ((), ())),
                           preferred_element_type=F32)


def _d01(a, b):
    return lax.dot_general(a, b, (((0,), (1,)), ((), ())),
                           preferred_element_type=F32)


def _d10(a, b):
    return lax.dot_general(a, b, (((1,), (0,)), ((), ())),
                           preferred_element_type=F32)


def _t0_body(xin_ref, wat, wbt, wag, wbg, bt, bg, at_o, bt_o, ag_o, bg_o):
    X = xin_ref[...]
    at_o[...] = _d00(X, wat[...]) + bt[...]
    bt_o[...] = _d00(X, wbt[...]).astype(BF16)
    ag_o[...] = _d00(X, wag[...]) + bg[...]
    bg_o[...] = _d00(X, wbg[...]).astype(BF16)


def _t0(xinT, WAt, WBt, WAg, WBg, bt, bg):
    co = WAt.shape[1]
    return pl.pallas_call(
        _t0_body,
        grid=(NB,),
        in_specs=[_nspec(xinT.shape[0]), _wspec(WAt.shape), _wspec(WBt.shape),
                  _wspec(WAg.shape), _wspec(WBg.shape), _wspec(bt.shape),
                  _wspec(bg.shape)],
        out_specs=[_mspec(co)] * 4,
        out_shape=[_sds((NP, co), F32), _sds((NP, co), BF16),
                   _sds((NP, co), F32), _sds((NP, co), BF16)],
    )(xinT, WAt, WBt, WAg, WBg, bt, bg)


def _tmid_body(at, ct, ag, cg, wm, bm, wat, bt2, wbt, wag, bg2, wbg,
               x_o, a2t_o, b2t_o, a2g_o, b2g_o):
    h = jnp.concatenate(
        [jnp.maximum(at[...] + ct[...].astype(F32), 0.0),
         jnp.maximum(ag[...] + cg[...].astype(F32), 0.0)], axis=1)
    xn = jnp.maximum(_d01(wm[...], h) + bm[...], 0.0)
    x_o[...] = xn
    a2t_o[...] = _d00(xn, wat[...]) + bt2[...]
    b2t_o[...] = _d00(xn, wbt[...]).astype(BF16)
    a2g_o[...] = _d00(xn, wag[...]) + bg2[...]
    b2g_o[...] = _d00(xn, wbg[...]).astype(BF16)


def _tmid(at, ct, ag, cg, wm, bm, wat, bt2, wbt, wag, bg2, wbg):
    ci = at.shape[1]
    cx = wm.shape[1]
    co = wat.shape[1]
    return pl.pallas_call(
        _tmid_body,
        grid=(NB,),
        in_specs=[_mspec(ci)] * 4 + [_wspec(wm.shape), _wspec(bm.shape),
                  _wspec(wat.shape), _wspec(bt2.shape), _wspec(wbt.shape),
                  _wspec(wag.shape), _wspec(bg2.shape), _wspec(wbg.shape)],
        out_specs=[_nspec(cx)] + [_mspec(co)] * 4,
        out_shape=[_sds((cx, NP), F32), _sds((NP, co), F32),
                   _sds((NP, co), BF16), _sds((NP, co), F32),
                   _sds((NP, co), BF16)],
    )(at, ct, ag, cg, wm, bm, wat, bt2, wbt, wag, bg2, wbg)


def _t3_body(at, ct, ag, cg, wm3, bm3, x1, x2, wg1, wg2, wg3, bglb, bb_ref,
             x3_o, xg_o):
    i = pl.program_id(0)
    h = jnp.concatenate(
        [jnp.maximum(at[...] + ct[...].astype(F32), 0.0),
         jnp.maximum(ag[...] + cg[...].astype(F32), 0.0)], axis=1)
    x3 = jnp.maximum(_d01(wm3[...], h) + bm3[...], 0.0)
    x3_o[...] = x3
    x4 = jnp.maximum(_d10(wg1[...], x1[...]) + _d10(wg2[...], x2[...]) +
                     _d10(wg3[...], x3) + bglb[...], 0.0)

    @pl.when(i == 0)
    def _():
        xg_o[...] = jnp.full((1024, 128), NEG, F32)

    bb = bb_ref[...]
    lane = lax.broadcasted_iota(I32, (1024, 128), 1)
    acc = xg_o[...]
    for b in range(4):
        colmax = jnp.max(jnp.where(bb == b, x4, NEG), axis=1, keepdims=True)
        acc = jnp.maximum(acc, jnp.where(lane == b, colmax, NEG))
    xg_o[...] = acc


def _t3(at, ct, ag, cg, wm3, bm3, x1, x2, wg1, wg2, wg3, bglb, batchP):
    return pl.pallas_call(
        _t3_body,
        grid=(NB,),
        in_specs=[_mspec(256)] * 4 + [_wspec(wm3.shape), _wspec(bm3.shape),
                  _nspec(64), _nspec(256), _wspec(wg1.shape),
                  _wspec(wg2.shape), _wspec(wg3.shape), _wspec(bglb.shape),
                  _nspec(1)],
        out_specs=[_nspec(512), pl.BlockSpec((1024, 128), lambda i: (0, 0))],
        out_shape=[_sds((512, NP), F32), _sds((1024, 128), F32)],
    )(at, ct, ag, cg, wm3, bm3, x1, x2, wg1, wg2, wg3, bglb, batchP)


def _t4_body(xgm_ref, bb_ref, xin, x1, x2, x3, wg, wxin, w1, w2, w3, btr,
             sig_o, den_o):
    i = pl.program_id(0)
    xgm = xgm_ref[...]
    xg = jnp.where(xgm > -1e29, xgm, 0.0)
    bb = bb_ref[...]
    xgb = jnp.zeros((1024, BN), F32)
    for b in range(4):
        xgb = xgb + jnp.where(bb == b, xg[:, b:b + 1], 0.0)
    z = (_d10(wg[...], xgb) + _d10(wxin[...], xin[...]) +
         _d10(w1[...], x1[...]) + _d10(w2[...], x2[...]) +
         _d10(w3[...], x3[...]) + btr[...])
    sig = 1.0 / (1.0 + jnp.exp(-z))
    sig_o[...] = sig

    @pl.when(i == 0)
    def _():
        den_o[...] = jnp.zeros((24, 128), F32)

    lane = lax.broadcasted_iota(I32, (24, 128), 1)
    acc = den_o[...]
    for b in range(4):
        colsum = jnp.sum(jnp.where(bb == b, sig, 0.0), axis=1, keepdims=True)
        acc = acc + jnp.where(lane == b, colsum, 0.0)
    den_o[...] = acc


def _t4(xgmax, batchP, xinT, x1, x2, x3, wg, wxin, w1, w2, w3, btr):
    return pl.pallas_call(
        _t4_body,
        grid=(NB,),
        in_specs=[pl.BlockSpec((1024, 128), lambda i: (0, 0)), _nspec(1),
                  _nspec(8), _nspec(64), _nspec(256), _nspec(512),
                  _wspec(wg.shape), _wspec(wxin.shape), _wspec(w1.shape),
                  _wspec(w2.shape), _wspec(w3.shape), _wspec(btr.shape)],
        out_specs=[_nspec(24), pl.BlockSpec((24, 128), lambda i: (0, 0))],
        out_shape=[_sds((24, NP), F32), _sds((24, 128), F32)],
    )(xgmax, batchP, xinT, x1, x2, x3, wg, wxin, w1, w2, w3, btr)


def _t5_body(sig_ref, den_ref, bb_ref, pos_ref, w_o, outs_o):
    i = pl.program_id(0)
    den = den_ref[...]
    recip = 1.0 / (den + 1e-9)
    bb = bb_ref[...]
    rsel = jnp.zeros((24, BN), F32)
    for b in range(4):
        rsel = rsel + jnp.where(bb == b, recip[:, b:b + 1], 0.0)
    w = sig_ref[...] * rsel
    w_o[...] = w

    @pl.when(i == 0)
    def _():
        outs_o[...] = jnp.zeros((96, 128), F32)

    acc = outs_o[...]
    pos = pos_ref[...]
    parts = []
    for b in range(4):
        mw = jnp.where(bb == b, w, 0.0)
        parts.append(_d10(mw, pos))
    outs_o[...] = acc + jnp.concatenate(parts, axis=0)


def _t5(sigT, den, batchP, pos128):
    return pl.pallas_call(
        _t5_body,
        grid=(NB,),
        in_specs=[_nspec(24), pl.BlockSpec((24, 128), lambda i: (0, 0)),
                  _nspec(1), pl.BlockSpec((BN, 128), lambda i: (i, 0))],
        out_specs=[_nspec(24), pl.BlockSpec((96, 128), lambda i: (0, 0))],
        out_shape=[_sds((24, NP), F32), _sds((96, 128), F32)],
    )(sigT, den, batchP, pos128)


def _sc_partition(E):
    CAPP = E + 2 * FL
    nchunk = E // ECH
    mesh = plsc.VectorSubcoreMesh(core_axis_name="c", subcore_axis_name="s")

    @functools.partial(
        pl.kernel,
        out_type=[_sds((32 * CAPP,), I32), _sds((32, 16), I32)],
        mesh=mesh,
        compiler_params=pltpu.CompilerParams(needs_layout_passes=False),
        scratch_types=[
            pltpu.VMEM((LB,), I32),
            pltpu.VMEM((2 * ECH,), I32),
            pltpu.VMEM((2 * ECH,), I32),
            pltpu.VMEM((16,), I32),
            pltpu.SMEM((2,), I32),
            pltpu.SemaphoreType.DMA,
            pltpu.SemaphoreType.DMA,
        ],
    )
    def k(srcl, dstl, pk_out, cnt_out, lbuf, sring, dring, cbuf, sref,
          sem0, sem1):
        wid = lax.axis_index("s") * 2 + lax.axis_index("c")
        lo = wid * 320
        base = wid * CAPP
        iota16 = lax.iota(I32, 16)
        sems = [sem0, sem1]

        def start(cidx, b):
            pltpu.async_copy(srcl.at[pl.ds(cidx * ECH, ECH)],
                             sring.at[pl.ds(b * ECH, ECH)], sems[b])
            pltpu.async_copy(dstl.at[pl.ds(cidx * ECH, ECH)],
                             dring.at[pl.ds(b * ECH, ECH)], sems[b])

        def wait(cidx, b):
            pltpu.make_async_copy(srcl.at[pl.ds(cidx * ECH, ECH)],
                                  sring.at[pl.ds(b * ECH, ECH)],
                                  sems[b]).wait()
            pltpu.make_async_copy(dstl.at[pl.ds(cidx * ECH, ECH)],
                                  dring.at[pl.ds(b * ECH, ECH)],
                                  sems[b]).wait()

        sref[0] = 0
        sref[1] = 0
        start(0, 0)

        def pair_body(g, _):
            for b in range(2):
                cidx = g * 2 + b
                wait(cidx, b)

                @pl.when(cidx + 1 < nchunk)
                def _():
                    start(cidx + 1, 1 - b)

                def vec_body(v, _):
                    d = dring[pl.ds(b * ECH + v * 16, 16)]
                    s = sring[pl.ds(b * ECH + v * 16, 16)]
                    m = jnp.logical_and(d >= lo, d < lo + 320)
                    pk = s * 512 + (d - lo)
                    cnt = sref[0]
                    pref = plsc.cumsum(m.astype(I32))
                    plsc.store_scatter(lbuf, [cnt + pref - 1], pk, mask=m)
                    sref[0] = cnt + plsc.all_reduce_population_count(m)[0]
                    return 0

                lax.fori_loop(0, ECH // 16, vec_body, 0)

                @pl.when(sref[0] >= FL)
                def _():
                    off = pl.multiple_of(base + sref[1], 8)
                    pltpu.sync_copy(lbuf.at[pl.ds(0, FL)],
                                    pk_out.at[pl.ds(off, FL)])

                    def mv(j, _):
                        lbuf[pl.ds(j * 16, 16)] = lbuf[pl.ds(FL + j * 16, 16)]
                        return 0

                    lax.fori_loop(0, (LB - FL) // 16, mv, 0)
                    sref[0] = sref[0] - FL
                    sref[1] = sref[1] + FL
            return 0

        lax.fori_loop(0, nchunk // 2, pair_body, 0)

        cnt = sref[0]

        def dum(j, _):
            plsc.store_scatter(lbuf, [cnt + iota16 + j * 16],
                               jnp.full((16,), 320, I32))
            return 0

        lax.fori_loop(0, FL // 16, dum, 0)
        off = pl.multiple_of(base + sref[1], 8)
        pltpu.sync_copy(lbuf.at[pl.ds(0, FL)],
                        pk_out.at[pl.ds(off, FL)])

        @pl.when(cnt > FL)
        def _():
            pltpu.sync_copy(lbuf.at[pl.ds(FL, FL)],
                            pk_out.at[pl.ds(off + FL, FL)])

        cntp = ((cnt + FL - 1) // FL) * FL
        cbuf[...] = jnp.full((16,), cntp, I32)
        pltpu.sync_copy(cbuf, cnt_out.at[wid])

    return k


def _sc_agg(ch, E):
    CAPP = E + 2 * FL
    CH32 = ch // 32
    mesh = plsc.VectorSubcoreMesh(core_axis_name="c", subcore_axis_name="s")

    @functools.partial(
        pl.kernel,
        out_type=_sds((NP, ch), BF16),
        mesh=mesh,
        compiler_params=pltpu.CompilerParams(needs_layout_passes=False,
                                             use_tc_tiling_on_sc=False),
        scratch_types=[
            pltpu.VMEM((ROWS, ch), BF16),
            pltpu.VMEM((GB, ch), BF16),
            pltpu.VMEM((GB, ch), BF16),
            pltpu.VMEM((GB,), I32),
            pltpu.VMEM((GB,), I32),
            pltpu.VMEM((GB,), I32),
            pltpu.VMEM((GB,), I32),
            pltpu.VMEM((GB,), I32),
            pltpu.VMEM((GB,), I32),
            pltpu.VMEM((16,), I32),
            pltpu.SemaphoreType.DMA,
            pltpu.SemaphoreType.DMA,
            pltpu.SemaphoreType.DMA,
            pltpu.SemaphoreType.DMA,
        ],
    )
    def k(pkl, cntl, tblh, cout, acc, rows0, rows1, pk0, pk1, idx0, idx1,
          dl0, dl1, cbuf, spk0, spk1, sg0, sg1):
        wid = lax.axis_index("s") * 2 + lax.axis_index("c")
        lo = wid * 320
        base = wid * CAPP
        rows = [rows0, rows1]
        pkb = [pk0, pk1]
        idxb = [idx0, idx1]
        dlb = [dl0, dl1]
        spk = [spk0, spk1]
        sg = [sg0, sg1]

        pltpu.sync_copy(cntl.at[wid], cbuf)
        cntp = cbuf[...][0]
        nb = cntp // GB

        def initb(r, _):
            for c in range(CH32):
                acc[r, pl.ds(c * 32, 32)] = jnp.full((32,), NEG, BF16)
            return 0

        lax.fori_loop(0, ROWS, initb, 0)

        def pk_start(i, b):
            o = pl.multiple_of(base + i * GB, 8)
            pltpu.async_copy(pkl.at[pl.ds(o, GB)], pkb[b], spk[b])

        def pk_wait(i, b):
            o = pl.multiple_of(base + i * GB, 8)
            pltpu.make_async_copy(pkl.at[pl.ds(o, GB)], pkb[b],
                                  spk[b]).wait()

        def build(b):
            for v in range(GB // 16):
                p = pkb[b][pl.ds(v * 16, 16)]
                idxb[b][pl.ds(v * 16, 16)] = lax.shift_right_logical(p, 9)
                dlb[b][pl.ds(v * 16, 16)] = lax.bitwise_and(p, 511)

        def g_start(b):
            pltpu.async_copy(tblh.at[idxb[b]], rows[b], sg[b])

        def g_wait(b):
            pltpu.make_async_copy(tblh.at[idxb[b]], rows[b], sg[b]).wait()

        @pl.when(nb > 0)
        def _():
            pk_start(0, 0)
            pk_wait(0, 0)
            build(0)
            g_start(0)

            @pl.when(nb > 1)
            def _():
                pk_start(1, 1)

            def pair_body(g2, _):
                for b in range(2):
                    i = g2 * 2 + b

                    @pl.when(i + 1 < nb)
                    def _():
                        pk_wait(i + 1, 1 - b)
                        build(1 - b)
                        g_start(1 - b)

                    @pl.when(i + 2 < nb)
                    def _():
                        pk_start(i + 2, b)

                    g_wait(b)

                    def vgroup(vv, _):
                        dlv = dlb[b][pl.ds(vv * 16, 16)]
                        for j in range(16):
                            dloc = dlv[j]
                            e = vv * 16 + j
                            for c in range(CH32):
                                r = rows[b][e, pl.ds(c * 32, 32)]
                                av = acc[dloc, pl.ds(c * 32, 32)]
                                acc[dloc, pl.ds(c * 32, 32)] = (
                                    jnp.maximum(av, r))
                        return 0

                    lax.fori_loop(0, GB // 16, vgroup, 0)
                return 0

            lax.fori_loop(0, nb // 2, pair_body, 0)

        pltpu.sync_copy(acc.at[pl.ds(0, 320)], cout.at[pl.ds(lo, 320)])

    return k


def _split_conv_w(W, b):
    ci = W.shape[0] // 2
    Wi, Wj = W[:ci], W[ci:]
    WA = Wi - Wj
    WB = Wj
    if ci == 6:
        WA = jnp.pad(WA, ((0, 2), (0, 0)))
        WB = jnp.pad(WB, ((0, 2), (0, 0)))
    return WA, WB, b[None, :]


def kernel(pos, x, tpl_edge_index, geo_edge_index, batch,
           Wt1, bt1, Wg1, bg1, Wm1, bm1,
           Wt2, bt2, Wg2, bg2, Wm2, bm2,
           Wt3, bt3, Wg3, bg3, Wm3, bm3,
           Wglb, bglb, Wtr, btr):
    N = pos.shape[0]
    E = tpl_edge_index.shape[1]

    xin = jnp.concatenate([pos, x], axis=1)
    xinT = jnp.zeros((8, NP), F32).at[:6, :N].set(xin.T)
    batchP = jnp.full((1, NP), 4, I32).at[0, :N].set(batch)
    pos128 = jnp.zeros((NP, 128), F32).at[:N, :3].set(pos)
    ts, td = tpl_edge_index[0], tpl_edge_index[1]
    gs, gd = geo_edge_index[0], geo_edge_index[1]

    WAt1, WBt1, bt1c = _split_conv_w(Wt1, bt1)
    WAg1, WBg1, bg1c = _split_conv_w(Wg1, bg1)
    WAt2, WBt2, bt2c = _split_conv_w(Wt2, bt2)
    WAg2, WBg2, bg2c = _split_conv_w(Wg2, bg2)
    WAt3, WBt3, bt3c = _split_conv_w(Wt3, bt3)
    WAg3, WBg3, bg3c = _split_conv_w(Wg3, bg3)
    bm1c, bm2c, bm3c = bm1[:, None], bm2[:, None], bm3[:, None]
    Wg1T = Wglb[:64].T
    Wg2T = Wglb[64:320].T
    Wg3T = Wglb[320:832].T
    bglbc = bglb[:, None]
    WtrT = Wtr.T
    Wtr_g = WtrT[:, :1024]
    Wtr_xin = jnp.pad(WtrT[:, 1024:1030], ((0, 0), (0, 2)))
    Wtr_1 = WtrT[:, 1030:1094]
    Wtr_2 = WtrT[:, 1094:1350]
    Wtr_3 = WtrT[:, 1350:1862]
    btrc = btr[:, None]

    part = _sc_partition(E)
    pk_t, cn_t = part(ts, td)
    pk_g, cn_g = part(gs, gd)
    agg32 = _sc_agg(32, E)
    agg128 = _sc_agg(128, E)
    agg256 = _sc_agg(256, E)

    At1, Bt1, Ag1, Bg1 = _t0(xinT, WAt1, WBt1, WAg1, WBg1, bt1c, bg1c)
    Ct1 = agg32(pk_t, cn_t, Bt1)
    Cg1 = agg32(pk_g, cn_g, Bg1)

    x1, At2, Bt2, Ag2, Bg2 = _tmid(At1, Ct1, Ag1, Cg1, Wm1, bm1c,
                                   WAt2, bt2c, WBt2, WAg2, bg2c, WBg2)
    Ct2 = agg128(pk_t, cn_t, Bt2)
    Cg2 = agg128(pk_g, cn_g, Bg2)

    x2, At3, Bt3, Ag3, Bg3 = _tmid(At2, Ct2, Ag2, Cg2, Wm2, bm2c,
                                   WAt3, bt3c, WBt3, WAg3, bg3c, WBg3)
    Ct3 = agg256(pk_t, cn_t, Bt3)
    Cg3 = agg256(pk_g, cn_g, Bg3)

    x3, xgmax = _t3(At3, Ct3, Ag3, Cg3, Wm3, bm3c, x1, x2,
                    Wg1T, Wg2T, Wg3T, bglbc, batchP)
    sigT, den = _t4(xgmax, batchP, xinT, x1, x2, x3,
                    Wtr_g, Wtr_xin, Wtr_1, Wtr_2, Wtr_3, btrc)
    wT, outsP = _t5(sigT, den, batchP, pos128)

    outs = outsP.reshape(4, 24, 128)[:, :, :3]
    w = wT[:, :N].T
    return outs, w


# speedup vs baseline: 9.7249x; 1.0219x over previous
import functools

import jax
import jax.numpy as jnp
from jax import lax
from jax.experimental import pallas as pl
from jax.experimental.pallas import tpu as pltpu
from jax.experimental.pallas import tpu_sc as plsc

F32 = jnp.float32
BF16 = jnp.bfloat16
I32 = jnp.int32
NEG = -1e30
NP = 10240
BN = 512
NB = NP // BN
ECH = 2000
FL = 2048
LB = 6144
GB = 128
ROWS = 328


def _sds(shape, dt):
    return jax.ShapeDtypeStruct(shape, dt)


def _nspec(c):
    return pl.BlockSpec((c, BN), lambda i: (0, i))


def _mspec(c):
    return pl.BlockSpec((BN, c), lambda i: (i, 0))


def _wspec(shape):
    return pl.BlockSpec(shape, lambda i: (0, 0))


def _d00(a, b):
    return lax.dot_general(a, b, (((0,), (0,)), ((), ())),
                           preferred_element_type=F32)


def _d01(a, b):
    return lax.dot_general(a, b, (((0,), (1,)), ((), ())),
                           preferred_element_type=F32)


def _d10(a, b):
    return lax.dot_general(a, b, (((1,), (0,)), ((), ())),
                           preferred_element_type=F32)


def _t0_body(xin_ref, wat, wbt, wag, wbg, bt, bg, at_o, bt_o, ag_o, bg_o):
    X = xin_ref[...]
    at_o[...] = _d00(X, wat[...]) + bt[...]
    bt_o[...] = _d00(X, wbt[...]).astype(BF16)
    ag_o[...] = _d00(X, wag[...]) + bg[...]
    bg_o[...] = _d00(X, wbg[...]).astype(BF16)


def _t0(xinT, WAt, WBt, WAg, WBg, bt, bg):
    co = WAt.shape[1]
    return pl.pallas_call(
        _t0_body,
        grid=(NB,),
        in_specs=[_nspec(xinT.shape[0]), _wspec(WAt.shape), _wspec(WBt.shape),
                  _wspec(WAg.shape), _wspec(WBg.shape), _wspec(bt.shape),
                  _wspec(bg.shape)],
        out_specs=[_mspec(co)] * 4,
        out_shape=[_sds((NP, co), F32), _sds((NP, co), BF16),
                   _sds((NP, co), F32), _sds((NP, co), BF16)],
    )(xinT, WAt, WBt, WAg, WBg, bt, bg)


def _tmid_body(at, ct, ag, cg, wm, bm, wat, bt2, wbt, wag, bg2, wbg,
               x_o, a2t_o, b2t_o, a2g_o, b2g_o):
    h = jnp.concatenate(
        [jnp.maximum(at[...] + ct[...].astype(F32), 0.0),
         jnp.maximum(ag[...] + cg[...].astype(F32), 0.0)],
        axis=1).astype(BF16)
    xn = jnp.maximum(_d01(wm[...], h) + bm[...], 0.0)
    x_o[...] = xn
    xh = xn.astype(BF16)
    a2t_o[...] = _d00(xh, wat[...]) + bt2[...]
    b2t_o[...] = _d00(xh, wbt[...]).astype(BF16)
    a2g_o[...] = _d00(xh, wag[...]) + bg2[...]
    b2g_o[...] = _d00(xh, wbg[...]).astype(BF16)


def _tmid(at, ct, ag, cg, wm, bm, wat, bt2, wbt, wag, bg2, wbg):
    ci = at.shape[1]
    cx = wm.shape[1]
    co = wat.shape[1]
    return pl.pallas_call(
        _tmid_body,
        grid=(NB,),
        in_specs=[_mspec(ci)] * 4 + [_wspec(wm.shape), _wspec(bm.shape),
                  _wspec(wat.shape), _wspec(bt2.shape), _wspec(wbt.shape),
                  _wspec(wag.shape), _wspec(bg2.shape), _wspec(wbg.shape)],
        out_specs=[_nspec(cx)] + [_mspec(co)] * 4,
        out_shape=[_sds((cx, NP), F32), _sds((NP, co), F32),
                   _sds((NP, co), BF16), _sds((NP, co), F32),
                   _sds((NP, co), BF16)],
    )(at, ct, ag, cg, wm, bm, wat, bt2, wbt, wag, bg2, wbg)


def _t3_body(at, ct, ag, cg, wm3, bm3, x1, x2, wg1, wg2, wg3, bglb, bb_ref,
             x3_o, xg_o):
    i = pl.program_id(0)
    h = jnp.concatenate(
        [jnp.maximum(at[...] + ct[...].astype(F32), 0.0),
         jnp.maximum(ag[...] + cg[...].astype(F32), 0.0)],
        axis=1).astype(BF16)
    x3 = jnp.maximum(_d01(wm3[...], h) + bm3[...], 0.0)
    x3_o[...] = x3
    x4 = jnp.maximum(_d10(wg1[...], x1[...].astype(BF16)) +
                     _d10(wg2[...], x2[...].astype(BF16)) +
                     _d10(wg3[...], x3.astype(BF16)) + bglb[...], 0.0)

    @pl.when(i == 0)
    def _():
        xg_o[...] = jnp.full((1024, 128), NEG, F32)

    bb = bb_ref[...]
    lane = lax.broadcasted_iota(I32, (1024, 128), 1)
    acc = xg_o[...]
    for b in range(4):
        colmax = jnp.max(jnp.where(bb == b, x4, NEG), axis=1, keepdims=True)
        acc = jnp.maximum(acc, jnp.where(lane == b, colmax, NEG))
    xg_o[...] = acc


def _t3(at, ct, ag, cg, wm3, bm3, x1, x2, wg1, wg2, wg3, bglb, batchP):
    return pl.pallas_call(
        _t3_body,
        grid=(NB,),
        in_specs=[_mspec(256)] * 4 + [_wspec(wm3.shape), _wspec(bm3.shape),
                  _nspec(64), _nspec(256), _wspec(wg1.shape),
                  _wspec(wg2.shape), _wspec(wg3.shape), _wspec(bglb.shape),
                  _nspec(1)],
        out_specs=[_nspec(512), pl.BlockSpec((1024, 128), lambda i: (0, 0))],
        out_shape=[_sds((512, NP), F32), _sds((1024, 128), F32)],
    )(at, ct, ag, cg, wm3, bm3, x1, x2, wg1, wg2, wg3, bglb, batchP)


def _t4_body(xgm_ref, bb_ref, xin, x1, x2, x3, wg, wxin, w1, w2, w3, btr,
             sig_o, den_o):
    i = pl.program_id(0)
    xgm = xgm_ref[...]
    xg = jnp.where(xgm > -1e29, xgm, 0.0)
    bb = bb_ref[...]
    xgb = jnp.zeros((1024, BN), F32)
    for b in range(4):
        xgb = xgb + jnp.where(bb == b, xg[:, b:b + 1], 0.0)
    z = (_d10(wg[...], xgb) + _d10(wxin[...], xin[...]) +
         _d10(w1[...], x1[...]) + _d10(w2[...], x2[...]) +
         _d10(w3[...], x3[...]) + btr[...])
    sig = 1.0 / (1.0 + jnp.exp(-z))
    sig_o[...] = sig

    @pl.when(i == 0)
    def _():
        den_o[...] = jnp.zeros((24, 128), F32)

    lane = lax.broadcasted_iota(I32, (24, 128), 1)
    acc = den_o[...]
    for b in range(4):
        colsum = jnp.sum(jnp.where(bb == b, sig, 0.0), axis=1, keepdims=True)
        acc = acc + jnp.where(lane == b, colsum, 0.0)
    den_o[...] = acc


def _t4(xgmax, batchP, xinT, x1, x2, x3, wg, wxin, w1, w2, w3, btr):
    return pl.pallas_call(
        _t4_body,
        grid=(NB,),
        in_specs=[pl.BlockSpec((1024, 128), lambda i: (0, 0)), _nspec(1),
                  _nspec(8), _nspec(64), _nspec(256), _nspec(512),
                  _wspec(wg.shape), _wspec(wxin.shape), _wspec(w1.shape),
                  _wspec(w2.shape), _wspec(w3.shape), _wspec(btr.shape)],
        out_specs=[_nspec(24), pl.BlockSpec((24, 128), lambda i: (0, 0))],
        out_shape=[_sds((24, NP), F32), _sds((24, 128), F32)],
    )(xgmax, batchP, xinT, x1, x2, x3, wg, wxin, w1, w2, w3, btr)


def _t5_body(sig_ref, den_ref, bb_ref, pos_ref, w_o, outs_o):
    i = pl.program_id(0)
    den = den_ref[...]
    recip = 1.0 / (den + 1e-9)
    bb = bb_ref[...]
    rsel = jnp.zeros((24, BN), F32)
    for b in range(4):
        rsel = rsel + jnp.where(bb == b, recip[:, b:b + 1], 0.0)
    w = sig_ref[...] * rsel
    w_o[...] = w

    @pl.when(i == 0)
    def _():
        outs_o[...] = jnp.zeros((96, 128), F32)

    acc = outs_o[...]
    pos = pos_ref[...]
    parts = []
    for b in range(4):
        mw = jnp.where(bb == b, w, 0.0)
        parts.append(_d10(mw, pos))
    outs_o[...] = acc + jnp.concatenate(parts, axis=0)


def _t5(sigT, den, batchP, pos128):
    return pl.pallas_call(
        _t5_body,
        grid=(NB,),
        in_specs=[_nspec(24), pl.BlockSpec((24, 128), lambda i: (0, 0)),
                  _nspec(1), pl.BlockSpec((BN, 128), lambda i: (i, 0))],
        out_specs=[_nspec(24), pl.BlockSpec((96, 128), lambda i: (0, 0))],
        out_shape=[_sds((24, NP), F32), _sds((96, 128), F32)],
    )(sigT, den, batchP, pos128)


def _sc_partition(E):
    CAPP = E + 2 * FL
    nchunk = E // ECH
    mesh = plsc.VectorSubcoreMesh(core_axis_name="c", subcore_axis_name="s")

    @functools.partial(
        pl.kernel,
        out_type=[_sds((32 * CAPP,), I32), _sds((32, 16), I32)],
        mesh=mesh,
        compiler_params=pltpu.CompilerParams(needs_layout_passes=False),
        scratch_types=[
            pltpu.VMEM((LB,), I32),
            pltpu.VMEM((2 * ECH,), I32),
            pltpu.VMEM((2 * ECH,), I32),
            pltpu.VMEM((16,), I32),
            pltpu.SMEM((2,), I32),
            pltpu.SemaphoreType.DMA,
            pltpu.SemaphoreType.DMA,
        ],
    )
    def k(srcl, dstl, pk_out, cnt_out, lbuf, sring, dring, cbuf, sref,
          sem0, sem1):
        wid = lax.axis_index("s") * 2 + lax.axis_index("c")
        lo = wid * 320
        base = wid * CAPP
        iota16 = lax.iota(I32, 16)
        sems = [sem0, sem1]

        def start(cidx, b):
            pltpu.async_copy(srcl.at[pl.ds(cidx * ECH, ECH)],
                             sring.at[pl.ds(b * ECH, ECH)], sems[b])
            pltpu.async_copy(dstl.at[pl.ds(cidx * ECH, ECH)],
                             dring.at[pl.ds(b * ECH, ECH)], sems[b])

        def wait(cidx, b):
            pltpu.make_async_copy(srcl.at[pl.ds(cidx * ECH, ECH)],
                                  sring.at[pl.ds(b * ECH, ECH)],
                                  sems[b]).wait()
            pltpu.make_async_copy(dstl.at[pl.ds(cidx * ECH, ECH)],
                                  dring.at[pl.ds(b * ECH, ECH)],
                                  sems[b]).wait()

        sref[0] = 0
        sref[1] = 0
        start(0, 0)

        def pair_body(g, _):
            for b in range(2):
                cidx = g * 2 + b
                wait(cidx, b)

                @pl.when(cidx + 1 < nchunk)
                def _():
                    start(cidx + 1, 1 - b)

                def vec_body(v, cntv):
                    d = dring[pl.ds(b * ECH + v * 16, 16)]
                    s = sring[pl.ds(b * ECH + v * 16, 16)]
                    m = jnp.logical_and(d >= lo, d < lo + 320)
                    pk = s * 512 + (d - lo)
                    pref = plsc.cumsum(m.astype(I32))
                    plsc.store_scatter(lbuf, [cntv + (pref - 1)], pk, mask=m)
                    return cntv + plsc.all_reduce_population_count(m)

                cntv0 = jnp.full((16,), sref[0], I32)
                cntv = lax.fori_loop(0, ECH // 16, vec_body, cntv0)
                sref[0] = cntv[0]

                @pl.when(sref[0] >= FL)
                def _():
                    off = pl.multiple_of(base + sref[1], 8)
                    pltpu.sync_copy(lbuf.at[pl.ds(0, FL)],
                                    pk_out.at[pl.ds(off, FL)])

                    def mv(j, _):
                        lbuf[pl.ds(j * 16, 16)] = lbuf[pl.ds(FL + j * 16, 16)]
                        return 0

                    lax.fori_loop(0, (LB - FL) // 16, mv, 0)
                    sref[0] = sref[0] - FL
                    sref[1] = sref[1] + FL
            return 0

        lax.fori_loop(0, nchunk // 2, pair_body, 0)

        cnt = sref[0]

        def dum(j, _):
            plsc.store_scatter(lbuf, [cnt + iota16 + j * 16],
                               jnp.full((16,), 320, I32))
            return 0

        lax.fori_loop(0, FL // 16, dum, 0)
        off = pl.multiple_of(base + sref[1], 8)
        pltpu.sync_copy(lbuf.at[pl.ds(0, FL)],
                        pk_out.at[pl.ds(off, FL)])

        @pl.when(cnt > FL)
        def _():
            pltpu.sync_copy(lbuf.at[pl.ds(FL, FL)],
                            pk_out.at[pl.ds(off + FL, FL)])

        cntp = ((cnt + FL - 1) // FL) * FL
        cbuf[...] = jnp.full((16,), cntp, I32)
        pltpu.sync_copy(cbuf, cnt_out.at[wid])

    return k


def _sc_agg(ch, E):
    CAPP = E + 2 * FL
    CH32 = ch // 32
    mesh = plsc.VectorSubcoreMesh(core_axis_name="c", subcore_axis_name="s")

    @functools.partial(
        pl.kernel,
        out_type=_sds((NP, ch), BF16),
        mesh=mesh,
        compiler_params=pltpu.CompilerParams(needs_layout_passes=False,
                                             use_tc_tiling_on_sc=False),
        scratch_types=[
            pltpu.VMEM((ROWS, ch), BF16),
            pltpu.VMEM((GB, ch), BF16),
            pltpu.VMEM((GB, ch), BF16),
            pltpu.VMEM((GB,), I32),
            pltpu.VMEM((GB,), I32),
            pltpu.VMEM((GB,), I32),
            pltpu.VMEM((GB,), I32),
            pltpu.VMEM((GB,), I32),
            pltpu.VMEM((GB,), I32),
            pltpu.VMEM((16,), I32),
            pltpu.SemaphoreType.DMA,
            pltpu.SemaphoreType.DMA,
            pltpu.SemaphoreType.DMA,
            pltpu.SemaphoreType.DMA,
        ],
    )
    def k(pkl, cntl, tblh, cout, acc, rows0, rows1, pk0, pk1, idx0, idx1,
          dl0, dl1, cbuf, spk0, spk1, sg0, sg1):
        wid = lax.axis_index("s") * 2 + lax.axis_index("c")
        lo = wid * 320
        base = wid * CAPP
        rows = [rows0, rows1]
        pkb = [pk0, pk1]
        idxb = [idx0, idx1]
        dlb = [dl0, dl1]
        spk = [spk0, spk1]
        sg = [sg0, sg1]

        pltpu.sync_copy(cntl.at[wid], cbuf)
        cntp = cbuf[...][0]
        nb = cntp // GB

        def initb(r, _):
            for c in range(CH32):
                acc[r, pl.ds(c * 32, 32)] = jnp.full((32,), NEG, BF16)
            return 0

        lax.fori_loop(0, ROWS, initb, 0)

        def pk_start(i, b):
            o = pl.multiple_of(base + i * GB, 8)
            pltpu.async_copy(pkl.at[pl.ds(o, GB)], pkb[b], spk[b])

        def pk_wait(i, b):
            o = pl.multiple_of(base + i * GB, 8)
            pltpu.make_async_copy(pkl.at[pl.ds(o, GB)], pkb[b],
                                  spk[b]).wait()

        def build(b):
            for v in range(GB // 16):
                p = pkb[b][pl.ds(v * 16, 16)]
                idxb[b][pl.ds(v * 16, 16)] = lax.shift_right_logical(p, 9)
                dlb[b][pl.ds(v * 16, 16)] = lax.bitwise_and(p, 511)

        def g_start(b):
            pltpu.async_copy(tblh.at[idxb[b]], rows[b], sg[b])

        def g_wait(b):
            pltpu.make_async_copy(tblh.at[idxb[b]], rows[b], sg[b]).wait()

        @pl.when(nb > 0)
        def _():
            pk_start(0, 0)
            pk_wait(0, 0)
            build(0)
            g_start(0)

            @pl.when(nb > 1)
            def _():
                pk_start(1, 1)

            def pair_body(g2, _):
                for b in range(2):
                    i = g2 * 2 + b

                    @pl.when(i + 1 < nb)
                    def _():
                        pk_wait(i + 1, 1 - b)
                        build(1 - b)
                        g_start(1 - b)

                    @pl.when(i + 2 < nb)
                    def _():
                        pk_start(i + 2, b)

                    g_wait(b)

                    def vgroup(vv, _):
                        dlv = dlb[b][pl.ds(vv * 16, 16)]
                        for j in range(16):
                            dloc = dlv[j]
                            e = vv * 16 + j
                            for c in range(CH32):
                                r = rows[b][e, pl.ds(c * 32, 32)]
                                av = acc[dloc, pl.ds(c * 32, 32)]
                                acc[dloc, pl.ds(c * 32, 32)] = (
                                    jnp.maximum(av, r))
                        return 0

                    lax.fori_loop(0, GB // 16, vgroup, 0)
                return 0

            lax.fori_loop(0, nb // 2, pair_body, 0)

        pltpu.sync_copy(acc.at[pl.ds(0, 320)], cout.at[pl.ds(lo, 320)])

    return k


def _split_conv_w(W, b):
    ci = W.shape[0] // 2
    Wi, Wj = W[:ci], W[ci:]
    WA = Wi - Wj
    WB = Wj
    if ci == 6:
        WA = jnp.pad(WA, ((0, 2), (0, 0)))
        WB = jnp.pad(WB, ((0, 2), (0, 0)))
    return WA, WB, b[None, :]


def kernel(pos, x, tpl_edge_index, geo_edge_index, batch,
           Wt1, bt1, Wg1, bg1, Wm1, bm1,
           Wt2, bt2, Wg2, bg2, Wm2, bm2,
           Wt3, bt3, Wg3, bg3, Wm3, bm3,
           Wglb, bglb, Wtr, btr):
    N = pos.shape[0]
    E = tpl_edge_index.shape[1]

    xin = jnp.concatenate([pos, x], axis=1)
    xinT = jnp.zeros((8, NP), F32).at[:6, :N].set(xin.T)
    batchP = jnp.full((1, NP), 4, I32).at[0, :N].set(batch)
    pos128 = jnp.zeros((NP, 128), F32).at[:N, :3].set(pos)
    ts, td = tpl_edge_index[0], tpl_edge_index[1]
    gs, gd = geo_edge_index[0], geo_edge_index[1]

    WAt1, WBt1, bt1c = _split_conv_w(Wt1, bt1)
    WAg1, WBg1, bg1c = _split_conv_w(Wg1, bg1)
    WAt2, WBt2, bt2c = _split_conv_w(Wt2, bt2)
    WAg2, WBg2, bg2c = _split_conv_w(Wg2, bg2)
    WAt3, WBt3, bt3c = _split_conv_w(Wt3, bt3)
    WAg3, WBg3, bg3c = _split_conv_w(Wg3, bg3)
    bm1c, bm2c, bm3c = bm1[:, None], bm2[:, None], bm3[:, None]
    Wg1T = Wglb[:64].T
    Wg2T = Wglb[64:320].T
    Wg3T = Wglb[320:832].T
    bglbc = bglb[:, None]
    WtrT = Wtr.T
    Wtr_g = WtrT[:, :1024]
    Wtr_xin = jnp.pad(WtrT[:, 1024:1030], ((0, 0), (0, 2)))
    Wtr_1 = WtrT[:, 1030:1094]
    Wtr_2 = WtrT[:, 1094:1350]
    Wtr_3 = WtrT[:, 1350:1862]
    btrc = btr[:, None]

    part = _sc_partition(E)
    pk_t, cn_t = part(ts, td)
    pk_g, cn_g = part(gs, gd)
    agg32 = _sc_agg(32, E)
    agg128 = _sc_agg(128, E)
    agg256 = _sc_agg(256, E)

    At1, Bt1, Ag1, Bg1 = _t0(xinT, WAt1, WBt1, WAg1, WBg1, bt1c, bg1c)
    Ct1 = agg32(pk_t, cn_t, Bt1)
    Cg1 = agg32(pk_g, cn_g, Bg1)

    x1, At2, Bt2, Ag2, Bg2 = _tmid(At1, Ct1, Ag1, Cg1,
                                   Wm1.astype(BF16), bm1c,
                                   WAt2.astype(BF16), bt2c,
                                   WBt2.astype(BF16),
                                   WAg2.astype(BF16), bg2c,
                                   WBg2.astype(BF16))
    Ct2 = agg128(pk_t, cn_t, Bt2)
    Cg2 = agg128(pk_g, cn_g, Bg2)

    x2, At3, Bt3, Ag3, Bg3 = _tmid(At2, Ct2, Ag2, Cg2,
                                   Wm2.astype(BF16), bm2c,
                                   WAt3.astype(BF16), bt3c,
                                   WBt3.astype(BF16),
                                   WAg3.astype(BF16), bg3c,
                                   WBg3.astype(BF16))
    Ct3 = agg256(pk_t, cn_t, Bt3)
    Cg3 = agg256(pk_g, cn_g, Bg3)

    x3, xgmax = _t3(At3, Ct3, Ag3, Cg3, Wm3.astype(BF16), bm3c, x1, x2,
                    Wg1T.astype(BF16), Wg2T.astype(BF16),
                    Wg3T.astype(BF16), bglbc, batchP)
    sigT, den = _t4(xgmax, batchP, xinT, x1, x2, x3,
                    Wtr_g, Wtr_xin, Wtr_1, Wtr_2, Wtr_3, btrc)
    wT, outsP = _t5(sigT, den, batchP, pos128)

    outs = outsP.reshape(4, 24, 128)[:, :, :3]
    w = wT[:, :N].T
    return outs, w
